# baseline (speedup 1.0000x reference)
import functools

import jax
import jax.numpy as jnp
from jax import lax
from jax.experimental import pallas as pl
from jax.experimental.pallas import tpu as pltpu
from jax.experimental.pallas import tpu_sc as plsc

B = 4096
L = 200
V = 1000000
EMB = 64
HID = 128

NC = 2
NS = 16
NW = NC * NS
BPW = B // NW
ULEN = (104, 96)
UOFF = (0, 104)
NSLOT = 6
UPR = 2
NUNIT = UPR * BPW
NMAIN = (NUNIT // NSLOT) * NSLOT

GBLK = 32768


def _g_body(tt_ref, w1_ref, out_ref):
    a = tt_ref[...].astype(jnp.bfloat16)
    w = w1_ref[...].astype(jnp.bfloat16)
    out_ref[...] = jax.lax.dot_general(
        a, w, (((0,), (0,)), ((), ())), preferred_element_type=jnp.float32
    )


_g_call = pl.pallas_call(
    _g_body,
    grid=((V + GBLK - 1) // GBLK,),
    in_specs=[
        pl.BlockSpec((EMB, GBLK), lambda i: (0, i)),
        pl.BlockSpec((EMB, HID), lambda i: (0, 0)),
    ],
    out_specs=pl.BlockSpec((GBLK, HID), lambda i: (i, 0)),
    out_shape=jax.ShapeDtypeStruct((V, HID), jnp.float32),
)

_mesh = plsc.VectorSubcoreMesh(core_axis_name="c", subcore_axis_name="s")


@functools.partial(
    pl.kernel,
    out_type=jax.ShapeDtypeStruct((B, HID), jnp.float32),
    mesh=_mesh,
    scratch_types=[
        pltpu.VMEM((BPW * L,), jnp.int32),
        pltpu.VMEM((NSLOT, ULEN[0], HID), jnp.float32),
        pltpu.VMEM((BPW, HID), jnp.float32),
    ] + [pltpu.SemaphoreType.DMA] * NSLOT,
)
def _pool_sum(x_hbm, g_hbm, out_hbm, idx_v, rows_v, acc_v, *sems):
    wid = lax.axis_index("s") * NC + lax.axis_index("c")
    base = wid * (BPW * L)
    pltpu.sync_copy(x_hbm.at[pl.ds(base, BPW * L)], idx_v)

    def issue(u, b):
        q = b % UPR
        ln = ULEN[q]
        off = (u >> 1) * L + UOFF[q]
        pltpu.async_copy(
            g_hbm.at[idx_v.at[pl.ds(off, ln)]],
            rows_v.at[b, pl.ds(0, ln)],
            sems[b],
        )

    def wait(b):
        ln = ULEN[b % UPR]
        pltpu.make_async_copy(
            g_hbm.at[pl.ds(0, ln)], rows_v.at[b, pl.ds(0, ln)], sems[b]
        ).wait()

    def reduce_unit(b, accs):
        ln = ULEN[b % UPR]

        def red(jo, accs):
            for u8 in range(8):
                j = jo * 8 + u8
                accs = tuple(
                    accs[k] + rows_v[b, j, pl.ds(k * 16, 16)]
                    for k in range(HID // 16)
                )
            return accs

        return lax.fori_loop(0, ln // 8, red, accs)

    for b in range(NSLOT - 1):
        issue(b, b)

    zeros = (jnp.zeros((16,), jnp.float32),) * (HID // 16)

    def store_row(row, accs):
        for k in range(HID // 16):
            acc_v[row, pl.ds(k * 16, 16)] = accs[k]

    def outer(g, _):
        u0 = g * NSLOT
        accs = zeros
        for b in range(NSLOT):
            u = u0 + b
            nxt = u + (NSLOT - 1)

            @pl.when(nxt < NUNIT)
            def _():
                issue(nxt, (b + NSLOT - 1) % NSLOT)

            wait(b)
            if b % UPR == 0:
                accs = reduce_unit(b, zeros)
            else:
                accs = reduce_unit(b, accs)
                store_row(u >> 1, accs)
        return 0

    lax.fori_loop(0, NMAIN // NSLOT, outer, 0)

    accs = zeros
    for u in range(NMAIN, NUNIT):
        b = u % NSLOT
        wait(b)
        if b % UPR == 0:
            accs = reduce_unit(b, zeros)
        else:
            accs = reduce_unit(b, accs)
            store_row(u >> 1, accs)

    pltpu.sync_copy(acc_v, out_hbm.at[pl.ds(wid * BPW, BPW)])


def _mlp_body(sums_ref, b1_ref, w2_ref, b2_ref, out_ref):
    h = jnp.maximum(sums_ref[...] * (1.0 / L) + b1_ref[...], 0.0)
    o = jnp.sum(h * w2_ref[...][:, 0], axis=1) + b2_ref[0]
    out_ref[...] = o


_mlp = pl.pallas_call(
    _mlp_body,
    out_shape=jax.ShapeDtypeStruct((B,), jnp.float32),
)


def kernel(x, table, W1, b1, W2, b2):
    tt = table.T
    g = _g_call(tt, W1)
    x_flat = x.reshape(-1).astype(jnp.int32)
    sums = _pool_sum(x_flat, g)
    return _mlp(sums, b1, W2, b2)


# speedup vs baseline: 1.0067x; 1.0067x over previous
import functools

import jax
import jax.numpy as jnp
from jax import lax
from jax.experimental import pallas as pl
from jax.experimental.pallas import tpu as pltpu
from jax.experimental.pallas import tpu_sc as plsc

B = 4096
L = 200
V = 1000000
EMB = 64
HID = 128

NC = 2
NS = 16
NW = NC * NS
BPW = B // NW
ULEN = (56, 48, 48, 48)
UOFF = (0, 56, 104, 152)
NSLOT = 8
UPR = 4
NUNIT = UPR * BPW

GBLK = 32768


def _g_body(tt_ref, w1_ref, out_ref):
    a = tt_ref[...].astype(jnp.bfloat16)
    w = w1_ref[...].astype(jnp.bfloat16)
    out_ref[...] = jax.lax.dot_general(
        a, w, (((0,), (0,)), ((), ())), preferred_element_type=jnp.float32
    )


_g_call = pl.pallas_call(
    _g_body,
    grid=((V + GBLK - 1) // GBLK,),
    in_specs=[
        pl.BlockSpec((EMB, GBLK), lambda i: (0, i)),
        pl.BlockSpec((EMB, HID), lambda i: (0, 0)),
    ],
    out_specs=pl.BlockSpec((GBLK, HID), lambda i: (i, 0)),
    out_shape=jax.ShapeDtypeStruct((V, HID), jnp.float32),
)

_mesh = plsc.VectorSubcoreMesh(core_axis_name="c", subcore_axis_name="s")


@functools.partial(
    pl.kernel,
    out_type=jax.ShapeDtypeStruct((B, HID), jnp.float32),
    mesh=_mesh,
    scratch_types=[
        pltpu.VMEM((BPW * L,), jnp.int32),
        pltpu.VMEM((NSLOT, ULEN[0], HID), jnp.float32),
        pltpu.VMEM((BPW, HID), jnp.float32),
    ] + [pltpu.SemaphoreType.DMA] * NSLOT,
)
def _pool_sum(x_hbm, g_hbm, out_hbm, idx_v, rows_v, acc_v, *sems):
    wid = lax.axis_index("s") * NC + lax.axis_index("c")
    base = wid * (BPW * L)
    pltpu.sync_copy(x_hbm.at[pl.ds(base, BPW * L)], idx_v)

    def issue(u, b):
        q = b % UPR
        ln = ULEN[q]
        off = (u >> 2) * L + UOFF[q]
        pltpu.async_copy(
            g_hbm.at[idx_v.at[pl.ds(off, ln)]],
            rows_v.at[b, pl.ds(0, ln)],
            sems[b],
        )

    def wait(b):
        ln = ULEN[b % UPR]
        pltpu.make_async_copy(
            g_hbm.at[pl.ds(0, ln)], rows_v.at[b, pl.ds(0, ln)], sems[b]
        ).wait()

    def reduce_unit(b, accs):
        ln = ULEN[b % UPR]

        def red(jo, accs):
            for u8 in range(8):
                j = jo * 8 + u8
                accs = tuple(
                    accs[k] + rows_v[b, j, pl.ds(k * 16, 16)]
                    for k in range(HID // 16)
                )
            return accs

        return lax.fori_loop(0, ln // 8, red, accs)

    for b in range(NSLOT - 1):
        issue(b, b)

    zeros = (jnp.zeros((16,), jnp.float32),) * (HID // 16)

    def outer(g, _):
        u0 = g * NSLOT
        accs = zeros
        for b in range(NSLOT):
            u = u0 + b
            nxt = u + (NSLOT - 1)

            @pl.when(nxt < NUNIT)
            def _():
                issue(nxt, (b + NSLOT - 1) % NSLOT)

            wait(b)
            if b % UPR == 0:
                accs = reduce_unit(b, zeros)
            else:
                accs = reduce_unit(b, accs)
            if b % UPR == UPR - 1:
                row = u >> 2
                for k in range(HID // 16):
                    acc_v[row, pl.ds(k * 16, 16)] = accs[k]
        return 0

    lax.fori_loop(0, NUNIT // NSLOT, outer, 0)
    pltpu.sync_copy(acc_v, out_hbm.at[pl.ds(wid * BPW, BPW)])


def _mlp_body(sums_ref, b1_ref, w2_ref, b2_ref, out_ref):
    h = jnp.maximum(sums_ref[...] * (1.0 / L) + b1_ref[...], 0.0)
    o = jnp.sum(h * w2_ref[...][:, 0], axis=1) + b2_ref[0]
    out_ref[...] = o


_mlp = pl.pallas_call(
    _mlp_body,
    out_shape=jax.ShapeDtypeStruct((B,), jnp.float32),
)


def kernel(x, table, W1, b1, W2, b2):
    tt = table.T
    g = _g_call(tt, W1)
    x_flat = x.reshape(-1).astype(jnp.int32)
    sums = _pool_sum(x_flat, g)
    return _mlp(sums, b1, W2, b2)
